# baseline (speedup 1.0000x reference)
import functools

import jax
import jax.numpy as jnp
from jax import lax
from jax.experimental import pallas as pl
from jax.experimental.pallas import tpu as pltpu
from jax.experimental.pallas import tpu_sc as plsc

B = 4096
T = 200
D = 64
TP = T + 1
DP = D + 1

NC = 2
NS = 16
NW = NC * NS
BPW = B // NW

IDX_ROWS = 2
IDX_COLS = T // IDX_ROWS


def _sc_body(rid_hbm, table_hbm, pos_hbm, out_hbm, idx_v, rows_v, outb_v,
             pos_v, post_v, gsems, osems):
    wid = lax.axis_index("s") * NC + lax.axis_index("c")
    base = wid * BPW
    iota = lax.iota(jnp.int32, 16)
    zero16 = jnp.zeros((16,), jnp.int32)
    d_rows = [iota + d0 for d0 in range(0, D, 16)]

    t_offs = tuple(range(0, T - 16, 16)) + (T - 16,)
    t_rows = {t0: iota + t0 for t0 in t_offs}

    pltpu.sync_copy(rid_hbm.at[pl.ds(base, BPW)], idx_v)

    def start_gather(i, p):
        for c in range(IDX_ROWS):
            pltpu.async_copy(
                table_hbm.at[idx_v.at[i].at[c]],
                rows_v.at[p].at[pl.ds(c * IDX_COLS, IDX_COLS)],
                gsems.at[p])

    def wait_gather(i, p):
        for c in range(IDX_ROWS):
            pltpu.make_async_copy(
                table_hbm.at[idx_v.at[i].at[c]],
                rows_v.at[p].at[pl.ds(c * IDX_COLS, IDX_COLS)],
                gsems.at[p]).wait()

    def store_refs(b, p):
        return outb_v.at[p].at[:, pl.ds(0, T)], out_hbm.at[b]

    def wait_store(b, p):
        src, dst = store_refs(b, p)
        pltpu.make_async_copy(src, dst, osems.at[p]).wait()

    start_gather(0, 0)
    start_gather(1, 1)
    pltpu.sync_copy(pos_hbm, pos_v)

    def pos_body(d, carry):
        dcol = zero16 + d
        for t0 in t_offs:
            vals = pos_v[pl.ds(d * T + t0, 16)]
            plsc.store_scatter(post_v, [t_rows[t0], dcol], vals)
        return carry

    lax.fori_loop(0, D, pos_body, 0)

    def pair_body(j, carry):
        for p in range(2):
            i = 2 * j + p
            wait_gather(i, p)

            @pl.when(j > 0)
            def _():
                wait_store(base + i - 2, p)

            def t_body(t8, tcarry):
                tb = t8 * 8
                for dt in range(8):
                    t = tb + dt
                    tcol = zero16 + t
                    for c, d0 in enumerate(range(0, D, 16)):
                        vals = (rows_v[p, t, pl.ds(d0, 16)]
                                + post_v[t, pl.ds(d0, 16)])
                        plsc.store_scatter(outb_v.at[p], [d_rows[c], tcol],
                                           vals)
                return tcarry

            lax.fori_loop(0, T // 8, t_body, 0)

            @pl.when(j < BPW // 2 - 1)
            def _():
                start_gather(i + 2, p)

            src, dst = store_refs(base + i, p)
            pltpu.async_copy(src, dst, osems.at[p])
        return carry

    lax.fori_loop(0, BPW // 2, pair_body, 0)
    wait_store(base + BPW - 2, 0)
    wait_store(base + BPW - 1, 1)


def kernel(rid, table, pos):
    rid3 = rid.astype(jnp.int32).reshape(B, IDX_ROWS, IDX_COLS)
    pos_flat = pos.reshape(D * T)
    mesh = plsc.VectorSubcoreMesh(core_axis_name="c", subcore_axis_name="s",
                                  num_cores=NC, num_subcores=NS)
    k = functools.partial(
        pl.kernel,
        out_type=jax.ShapeDtypeStruct((B, D, T), jnp.float32),
        mesh=mesh,
        compiler_params=pltpu.CompilerParams(needs_layout_passes=False,
                                             use_tc_tiling_on_sc=False),
        scratch_types=[
            pltpu.VMEM((BPW, IDX_ROWS, IDX_COLS), jnp.int32),
            pltpu.VMEM((2, T, D), jnp.float32),
            pltpu.VMEM((2, D, TP), jnp.float32),
            pltpu.VMEM((D * T,), jnp.float32),
            pltpu.VMEM((T, DP), jnp.float32),
            pltpu.SemaphoreType.DMA((2,)),
            pltpu.SemaphoreType.DMA((2,)),
        ],
    )(_sc_body)
    return k(rid3, table, pos_flat)


# speedup vs baseline: 1.4396x; 1.4396x over previous
import functools

import jax
import jax.numpy as jnp
from jax import lax
from jax.experimental import pallas as pl
from jax.experimental.pallas import tpu as pltpu
from jax.experimental.pallas import tpu_sc as plsc

B = 4096
T = 200
D = 64

NC = 2
NS = 16
NW = NC * NS
BPW = B // NW

IDX_ROWS = 2
IDX_COLS = T // IDX_ROWS


def _sc_body(rid_hbm, table_hbm, post_hbm, out_hbm, idx_v, rows_v, post_v,
             gsems, osems):
    wid = lax.axis_index("s") * NC + lax.axis_index("c")
    base = wid * BPW

    pltpu.sync_copy(rid_hbm.at[pl.ds(base, BPW)], idx_v)

    def start_gather(i, p):
        for c in range(IDX_ROWS):
            pltpu.async_copy(
                table_hbm.at[idx_v.at[i].at[c]],
                rows_v.at[p].at[pl.ds(c * IDX_COLS, IDX_COLS)],
                gsems.at[p])

    def wait_gather(i, p):
        for c in range(IDX_ROWS):
            pltpu.make_async_copy(
                table_hbm.at[idx_v.at[i].at[c]],
                rows_v.at[p].at[pl.ds(c * IDX_COLS, IDX_COLS)],
                gsems.at[p]).wait()

    def wait_store(b, p):
        pltpu.make_async_copy(rows_v.at[p], out_hbm.at[b], osems.at[p]).wait()

    start_gather(0, 0)
    start_gather(1, 1)
    pltpu.sync_copy(post_hbm, post_v)

    def pair_body(j, carry):
        for p in range(2):
            i = 2 * j + p
            wait_gather(i, p)

            @pl.when(j > 0)
            def _():
                wait_store(base + i - 2, p)

            def t_body(t8, tcarry):
                tb = t8 * 8
                for dt in range(8):
                    t = tb + dt
                    for d0 in range(0, D, 16):
                        plsc.addupdate(rows_v.at[p].at[t].at[pl.ds(d0, 16)],
                                       post_v[t, pl.ds(d0, 16)])
                return tcarry

            lax.fori_loop(0, T // 8, t_body, 0)

            @pl.when(j < BPW // 2 - 1)
            def _():
                start_gather(i + 2, p)

            pltpu.async_copy(rows_v.at[p], out_hbm.at[base + i], osems.at[p])
        return carry

    lax.fori_loop(0, BPW // 2, pair_body, 0)
    wait_store(base + BPW - 2, 0)
    wait_store(base + BPW - 1, 1)


def kernel(rid, table, pos):
    rid3 = rid.astype(jnp.int32).reshape(B, IDX_ROWS, IDX_COLS)
    pos_t = jnp.transpose(pos.reshape(D, T))
    mesh = plsc.VectorSubcoreMesh(core_axis_name="c", subcore_axis_name="s",
                                  num_cores=NC, num_subcores=NS)
    k = functools.partial(
        pl.kernel,
        out_type=jax.ShapeDtypeStruct((B, T, D), jnp.float32),
        mesh=mesh,
        compiler_params=pltpu.CompilerParams(needs_layout_passes=False,
                                             use_tc_tiling_on_sc=False),
        scratch_types=[
            pltpu.VMEM((BPW, IDX_ROWS, IDX_COLS), jnp.int32),
            pltpu.VMEM((2, T, D), jnp.float32),
            pltpu.VMEM((T, D), jnp.float32),
            pltpu.SemaphoreType.DMA((2,)),
            pltpu.SemaphoreType.DMA((2,)),
        ],
    )(_sc_body)
    return jnp.transpose(k(rid3, table, pos_t), (0, 2, 1))


# speedup vs baseline: 1.4416x; 1.0014x over previous
import functools

import jax
import jax.numpy as jnp
from jax import lax
from jax.experimental import pallas as pl
from jax.experimental.pallas import tpu as pltpu
from jax.experimental.pallas import tpu_sc as plsc

B = 4096
T = 200
D = 64

NC = 2
NS = 16
NW = NC * NS
BPW = B // NW

IDX_SPLITS = ((0, 104), (104, 96))


def _sc_body(rid_hbm, table_hbm, post_hbm, out_hbm, idx_v, rows_v, post_v,
             gsems, osems):
    wid = lax.axis_index("s") * NC + lax.axis_index("c")
    base = wid * BPW

    pltpu.sync_copy(rid_hbm.at[pl.ds(base, BPW)], idx_v)

    def start_gather(i, p):
        for off, n in IDX_SPLITS:
            pltpu.async_copy(
                table_hbm.at[idx_v.at[i].at[pl.ds(off, n)]],
                rows_v.at[p].at[pl.ds(off, n)],
                gsems.at[p])

    def wait_gather(i, p):
        for off, n in IDX_SPLITS:
            pltpu.make_async_copy(
                table_hbm.at[idx_v.at[i].at[pl.ds(off, n)]],
                rows_v.at[p].at[pl.ds(off, n)],
                gsems.at[p]).wait()

    def wait_store(b, p):
        pltpu.make_async_copy(rows_v.at[p], out_hbm.at[b], osems.at[p]).wait()

    start_gather(0, 0)
    start_gather(1, 1)
    pltpu.sync_copy(post_hbm, post_v)

    def pair_body(j, carry):
        for p in range(2):
            i = 2 * j + p
            wait_gather(i, p)

            @pl.when(j > 0)
            def _():
                wait_store(base + i - 2, p)

            def t_body(t8, tcarry):
                tb = t8 * 8
                for dt in range(8):
                    t = tb + dt
                    for d0 in range(0, D, 16):
                        plsc.addupdate(rows_v.at[p].at[t].at[pl.ds(d0, 16)],
                                       post_v[t, pl.ds(d0, 16)])
                return tcarry

            lax.fori_loop(0, T // 8, t_body, 0)

            @pl.when(j < BPW // 2 - 1)
            def _():
                start_gather(i + 2, p)

            pltpu.async_copy(rows_v.at[p], out_hbm.at[base + i], osems.at[p])
        return carry

    lax.fori_loop(0, BPW // 2, pair_body, 0)
    wait_store(base + BPW - 2, 0)
    wait_store(base + BPW - 1, 1)


def kernel(rid, table, pos):
    rid32 = rid.astype(jnp.int32)
    pos_t = jnp.transpose(pos.reshape(D, T))
    mesh = plsc.VectorSubcoreMesh(core_axis_name="c", subcore_axis_name="s",
                                  num_cores=NC, num_subcores=NS)
    k = functools.partial(
        pl.kernel,
        out_type=jax.ShapeDtypeStruct((B, T, D), jnp.float32),
        mesh=mesh,
        compiler_params=pltpu.CompilerParams(needs_layout_passes=False,
                                             use_tc_tiling_on_sc=False),
        scratch_types=[
            pltpu.VMEM((BPW, T), jnp.int32),
            pltpu.VMEM((2, T, D), jnp.float32),
            pltpu.VMEM((T, D), jnp.float32),
            pltpu.SemaphoreType.DMA((2,)),
            pltpu.SemaphoreType.DMA((2,)),
        ],
    )(_sc_body)
    return jnp.transpose(k(rid32, table, pos_t), (0, 2, 1))
